# Initial kernel scaffold; baseline (speedup 1.0000x reference)
import jax
import jax.numpy as jnp
from jax.experimental import pallas as pl


def kernel(x, edge0, edge1, edge2, dec_edges, l1_W, l1_b, l2_W, l2_b, mlp_W1, mlp_b1, mlp_W2, mlp_b2):
    raise NotImplementedError("write your pallas kernel here")


# speedup vs baseline: 2.4885x; 2.4885x over previous
import functools

import jax
import jax.numpy as jnp
from jax import lax
from jax.experimental import pallas as pl
from jax.experimental.pallas import tpu as pltpu
from jax.experimental.pallas import tpu_sc as plsc

N = 50000
F = 128
E = 200000
ED = 100000
NREL = 3

NC = 2
NS = 16
L = 16

E_PAD = 200704
EPT = 12544
NBAT = EPT // 128
PADID = N
AROWS = 50176
ASTRIPE = AROWS // NS
G = 8
GW = 16
ZROWS = 51200
ED_PAD = 102400
DPW = 3200


def _mesh():
    return plsc.VectorSubcoreMesh(core_axis_name="c", subcore_axis_name="s")


@functools.partial(
    pl.kernel,
    out_type=jax.ShapeDtypeStruct((NC, NREL, AROWS, GW), jnp.float32),
    mesh=_mesh(),
    compiler_params=pltpu.CompilerParams(use_tc_tiling_on_sc=False),
    scratch_types=[
        pltpu.VMEM((NBAT, 128), jnp.int32),
        pltpu.VMEM((128, GW), jnp.float32),
        pltpu.VMEM((196, GW), jnp.float32),
        pltpu.VMEM_SHARED((AROWS, GW), jnp.float32),
    ],
)
def _sc_deg(ec0, ec1, ec2, onesb, zerosb, degf, idx2d, onesv, zv, acc):
    core = lax.axis_index("c")
    sub = lax.axis_index("s")
    base0 = core * E_PAD + sub * EPT
    pltpu.sync_copy(onesb, onesv)
    pltpu.sync_copy(zerosb, zv)

    for r, ec in enumerate((ec0, ec1, ec2)):
        def ld(b, _):
            pltpu.sync_copy(ec.at[pl.ds(base0 + b * 128, 128)], idx2d.at[b])
            return 0

        lax.fori_loop(0, NBAT, ld, 0)
        for k in range(16):
            pltpu.sync_copy(zv, acc.at[pl.ds(sub * ASTRIPE + k * 196, 196)])
        plsc.subcore_barrier()

        def body(b, _):
            pltpu.sync_copy(onesv, acc.at[idx2d.at[b]], add=True)
            return 0

        lax.fori_loop(0, NBAT, body, 0)
        plsc.subcore_barrier()
        pltpu.sync_copy(
            acc.at[pl.ds(sub * ASTRIPE, ASTRIPE)],
            degf.at[core, r, pl.ds(sub * ASTRIPE, ASTRIPE)])
        plsc.subcore_barrier()


@functools.partial(
    pl.kernel,
    out_type=[jax.ShapeDtypeStruct((AROWS, F), jnp.float32)
              for _ in range(NREL)],
    mesh=_mesh(),
    compiler_params=pltpu.CompilerParams(use_tc_tiling_on_sc=False),
    scratch_types=[
        pltpu.VMEM((EPT,), jnp.int32),
        pltpu.VMEM((EPT,), jnp.int32),
        pltpu.VMEM((NBAT, 128), jnp.int32),
        pltpu.VMEM((128, GW), jnp.float32),
        pltpu.VMEM((196, GW), jnp.float32),
        pltpu.VMEM_SHARED((AROWS, GW), jnp.float32),
        pltpu.SemaphoreType.DMA,
    ],
)
def _sc_prop(zf0, zf1, zf2, es0, es1, es2, ed0, ed1, ed2, zerosb,
             a0, a1, a2, srcb, idxg, dst2d, rows, zv, acc, sem):
    core = lax.axis_index("c")
    sub = lax.axis_index("s")
    ebase = sub * EPT
    pltpu.sync_copy(zerosb, zv)

    for zf, es, ed, ag in ((zf0, es0, ed0, a0), (zf1, es1, ed1, a1),
                           (zf2, es2, ed2, a2)):
        pltpu.sync_copy(es.at[pl.ds(ebase, EPT)], srcb)

        def ld(b, _):
            pltpu.sync_copy(ed.at[pl.ds(ebase + b * 128, 128)], dst2d.at[b])
            return 0

        lax.fori_loop(0, NBAT, ld, 0)
        for ch in range(4):
            g = core * 4 + ch
            for k in range(16):
                pltpu.sync_copy(
                    zv, acc.at[pl.ds(sub * ASTRIPE + k * 196, 196)])

            def fl(k, _):
                idxg[pl.ds(k * L, L)] = srcb[pl.ds(k * L, L)] * G + g
                return 0

            lax.fori_loop(0, EPT // L, fl, 0)
            plsc.subcore_barrier()

            def body(b, _):
                pltpu.async_copy(
                    zf.at[idxg.at[pl.ds(b * 128, 128)]], rows, sem).wait()
                pltpu.sync_copy(rows, acc.at[dst2d.at[b]], add=True)
                return 0

            lax.fori_loop(0, NBAT, body, 0)
            plsc.subcore_barrier()
            obase = sub * ASTRIPE
            pltpu.sync_copy(
                acc.at[pl.ds(obase, ASTRIPE)],
                ag.at[pl.ds(obase, ASTRIPE), pl.ds(g * GW, GW)])
            plsc.subcore_barrier()


@functools.partial(
    pl.kernel,
    out_type=[jax.ShapeDtypeStruct((ED_PAD, F), jnp.float32)
              for _ in range(2)],
    mesh=_mesh(),
    scratch_types=[
        pltpu.VMEM((DPW,), jnp.int32),
        pltpu.VMEM((128, F), jnp.float32),
        pltpu.SemaphoreType.DMA,
    ],
)
def _sc_dec_gather(h2, dsrc, ddst, gs, gd, idxb, rows, sem):
    core = lax.axis_index("c")
    sub = lax.axis_index("s")
    base = (core * NS + sub) * DPW
    for idx_hbm, out in ((dsrc, gs), (ddst, gd)):
        pltpu.sync_copy(idx_hbm.at[pl.ds(base, DPW)], idxb)

        def body(b, _):
            pltpu.async_copy(
                h2.at[idxb.at[pl.ds(b * 128, 128)]], rows, sem).wait()
            pltpu.sync_copy(rows, out.at[pl.ds(base + b * 128, 128)])
            return 0

        lax.fori_loop(0, DPW // 128, body, 0)


RB = 2048
NBLK = ZROWS // RB


def _norms_body(degf_ref, norms_ref):
    deg = degf_ref[:, :, :, 0]
    norms_ref[...] = jnp.where(
        deg > 0, lax.rsqrt(jnp.maximum(deg, 1.0)), 0.0)


def _tc_norms(degf):
    return pl.pallas_call(
        _norms_body,
        grid=(NBLK,),
        in_specs=[pl.BlockSpec((NC, NREL, RB, GW), lambda i: (0, 0, i, 0))],
        out_specs=pl.BlockSpec((NC, NREL, RB), lambda i: (0, 0, i)),
        out_shape=jax.ShapeDtypeStruct((NC, NREL, AROWS), jnp.float32),
    )(degf)


def _transform1_body(x_ref, n_ref, w_ref, z0_ref, z1_ref, z2_ref):
    xb = x_ref[...]
    for r, zr in enumerate((z0_ref, z1_ref, z2_ref)):
        z = jnp.dot(xb, w_ref[r], preferred_element_type=jnp.float32)
        zr[...] = z * n_ref[0, r][:, None]


def _tc_transform1(x, norms, l1_W):
    return pl.pallas_call(
        _transform1_body,
        grid=(NBLK,),
        in_specs=[
            pl.BlockSpec((RB, F), lambda i: (i, 0)),
            pl.BlockSpec((NC, NREL, RB), lambda i: (0, 0, i)),
            pl.BlockSpec((NREL, F, F), lambda i: (0, 0, 0)),
        ],
        out_specs=[pl.BlockSpec((RB, F), lambda i: (i, 0))] * NREL,
        out_shape=[jax.ShapeDtypeStruct((ZROWS, F), jnp.float32)] * NREL,
    )(x, norms, l1_W)


def _combine2_body(a0, a1, a2, n_ref, b_ref, w_ref, z0_ref, z1_ref, z2_ref):
    h = (a0[...] * n_ref[1, 0][:, None]
         + a1[...] * n_ref[1, 1][:, None]
         + a2[...] * n_ref[1, 2][:, None]
         + 3.0 * b_ref[...])
    h = jnp.maximum(h, 0.0)
    for r, zr in enumerate((z0_ref, z1_ref, z2_ref)):
        z = jnp.dot(h, w_ref[r], preferred_element_type=jnp.float32)
        zr[...] = z * n_ref[0, r][:, None]


def _tc_combine_transform2(a0, a1, a2, norms, l1_b, l2_W):
    return pl.pallas_call(
        _combine2_body,
        grid=(NBLK,),
        in_specs=[
            pl.BlockSpec((RB, F), lambda i: (i, 0)),
            pl.BlockSpec((RB, F), lambda i: (i, 0)),
            pl.BlockSpec((RB, F), lambda i: (i, 0)),
            pl.BlockSpec((NC, NREL, RB), lambda i: (0, 0, i)),
            pl.BlockSpec((1, F), lambda i: (0, 0)),
            pl.BlockSpec((NREL, F, F), lambda i: (0, 0, 0)),
        ],
        out_specs=[pl.BlockSpec((RB, F), lambda i: (i, 0))] * NREL,
        out_shape=[jax.ShapeDtypeStruct((ZROWS, F), jnp.float32)] * NREL,
    )(a0, a1, a2, norms, l1_b.reshape(1, F), l2_W)


def _final_h2_body(a0, a1, a2, n_ref, b_ref, h2_ref):
    h2_ref[...] = (a0[...] * n_ref[1, 0][:, None]
                   + a1[...] * n_ref[1, 1][:, None]
                   + a2[...] * n_ref[1, 2][:, None]
                   + 3.0 * b_ref[...])


def _tc_final_h2(a0, a1, a2, norms, l2_b):
    return pl.pallas_call(
        _final_h2_body,
        grid=(NBLK,),
        in_specs=[
            pl.BlockSpec((RB, F), lambda i: (i, 0)),
            pl.BlockSpec((RB, F), lambda i: (i, 0)),
            pl.BlockSpec((RB, F), lambda i: (i, 0)),
            pl.BlockSpec((NC, NREL, RB), lambda i: (0, 0, i)),
            pl.BlockSpec((1, F), lambda i: (0, 0)),
        ],
        out_specs=pl.BlockSpec((RB, F), lambda i: (i, 0)),
        out_shape=jax.ShapeDtypeStruct((ZROWS, F), jnp.float32),
    )(a0, a1, a2, norms, l2_b.reshape(1, F))


MB = 2048


def _mlp_body(gs_ref, gd_ref, w1a_ref, w1b_ref, b1_ref, w2_ref, b2_ref, o_ref):
    e = (jnp.dot(gs_ref[...], w1a_ref[...], preferred_element_type=jnp.float32)
         + jnp.dot(gd_ref[...], w1b_ref[...],
                   preferred_element_type=jnp.float32)
         + b1_ref[...])
    e = jnp.maximum(e, 0.0)
    o_ref[...] = jnp.dot(
        e, w2_ref[...], preferred_element_type=jnp.float32) + b2_ref[...]


def _tc_mlp(gs, gd, w1a, w1b, b1, w2, b2):
    ncls = w2.shape[1]
    return pl.pallas_call(
        _mlp_body,
        grid=(ED_PAD // MB,),
        in_specs=[
            pl.BlockSpec((MB, F), lambda i: (i, 0)),
            pl.BlockSpec((MB, F), lambda i: (i, 0)),
            pl.BlockSpec((F, F), lambda i: (0, 0)),
            pl.BlockSpec((F, F), lambda i: (0, 0)),
            pl.BlockSpec((1, F), lambda i: (0, 0)),
            pl.BlockSpec((F, ncls), lambda i: (0, 0)),
            pl.BlockSpec((1, ncls), lambda i: (0, 0)),
        ],
        out_specs=pl.BlockSpec((MB, ncls), lambda i: (i, 0)),
        out_shape=jax.ShapeDtypeStruct((ED_PAD, ncls), jnp.float32),
    )(gs, gd, w1a, w1b, b1.reshape(1, F), w2, b2.reshape(1, ncls))


def kernel(x, edge0, edge1, edge2, dec_edges, l1_W, l1_b, l2_W, l2_b,
           mlp_W1, mlp_b1, mlp_W2, mlp_b2):
    padv = jnp.full((E_PAD - E,), PADID, jnp.int32)
    es = [jnp.concatenate([e[0], padv]) for e in (edge0, edge1, edge2)]
    ed = [jnp.concatenate([e[1], padv]) for e in (edge0, edge1, edge2)]
    ec = [jnp.concatenate([s, d]) for s, d in zip(es, ed)]
    pad_e = jnp.zeros((ED_PAD - ED,), jnp.int32)
    dsrc = jnp.concatenate([dec_edges[0], pad_e])
    ddst = jnp.concatenate([dec_edges[1], pad_e])
    onesb = jnp.ones((128, GW), jnp.float32)
    zerosb = jnp.zeros((196, GW), jnp.float32)

    degf = _sc_deg(ec[0], ec[1], ec[2], onesb, zerosb)
    norms = _tc_norms(degf)
    z1 = _tc_transform1(x, norms, l1_W)
    zf1 = [z.reshape(G * ZROWS, GW) for z in z1]
    a1 = _sc_prop(zf1[0], zf1[1], zf1[2], es[0], es[1], es[2],
                  ed[0], ed[1], ed[2], zerosb)
    z2 = _tc_combine_transform2(a1[0], a1[1], a1[2], norms, l1_b, l2_W)
    zf2 = [z.reshape(G * ZROWS, GW) for z in z2]
    a2 = _sc_prop(zf2[0], zf2[1], zf2[2], es[0], es[1], es[2],
                  ed[0], ed[1], ed[2], zerosb)
    h2 = _tc_final_h2(a2[0], a2[1], a2[2], norms, l2_b)
    gs, gd = _sc_dec_gather(h2, dsrc, ddst)
    out = _tc_mlp(gs, gd, mlp_W1[:F], mlp_W1[F:], mlp_b1, mlp_W2, mlp_b2)
    return out[:ED]
